# Initial kernel scaffold; baseline (speedup 1.0000x reference)
import jax
import jax.numpy as jnp
from jax.experimental import pallas as pl


def kernel(x, table, W, b):
    raise NotImplementedError("write your pallas kernel here")


# speedup vs baseline: 3.2347x; 3.2347x over previous
import jax
import jax.numpy as jnp
from jax import lax
from jax.experimental import pallas as pl
from jax.experimental.pallas import tpu as pltpu
from jax.experimental.pallas import tpu_sc as plsc

_VOCAB = 1000000
_D = 64
_B = 16384
_L = 200

_BLK = 8000


def _mv_body(t_ref, w_ref, b_ref, o_ref):
    t = t_ref[...]
    w = w_ref[...]
    o_ref[...] = (jnp.dot(t, w, preferred_element_type=jnp.float32)
                  + b_ref[0, 0]) * (1.0 / _L)


def _matvec(table, w, b2):
    return pl.pallas_call(
        _mv_body,
        grid=(_VOCAB // _BLK,),
        in_specs=[
            pl.BlockSpec((_BLK, _D), lambda i: (i, 0)),
            pl.BlockSpec((_D, 1), lambda i: (0, 0)),
            pl.BlockSpec((1, 1), lambda i: (0, 0)),
        ],
        out_specs=pl.BlockSpec((_BLK, 1), lambda i: (i, 0)),
        out_shape=jax.ShapeDtypeStruct((_VOCAB, 1), jnp.float32),
    )(table, w, b2)


_NC = 2
_NS = 16
_NW = _NC * _NS
_ROWS_W = _B // _NW
_CHUNK_ROWS = 128
_CHUNK = _CHUNK_ROWS * _L
_NCHUNK = _ROWS_W // _CHUNK_ROWS


def _gather_body(v_hbm, xf_hbm, out_hbm, idx_v, val_v, out_v, sem):
    wid = lax.axis_index("s") * _NC + lax.axis_index("c")
    lane = lax.iota(jnp.int32, 16) * _L

    def chunk_body(c, carry):
        base = wid * (_ROWS_W * _L) + c * _CHUNK
        pltpu.sync_copy(xf_hbm.at[pl.ds(base, _CHUNK)], idx_v)
        pltpu.async_copy(v_hbm.at[idx_v], val_v, sem).wait()

        def grp(g, carry2):
            goff = g * (16 * _L)

            def jbody(j, accs):
                a0, a1 = accs
                off = goff + j * 8
                for u in range(8):
                    gth = plsc.load_gather(val_v, [lane + (off + u)])
                    if u % 2 == 0:
                        a0 = a0 + gth
                    else:
                        a1 = a1 + gth
                return (a0, a1)

            z = jnp.zeros((16,), jnp.float32)
            a0, a1 = lax.fori_loop(0, _L // 8, jbody, (z, z))
            out_v[pl.ds(c * _CHUNK_ROWS + g * 16, 16)] = a0 + a1
            return carry2

        return lax.fori_loop(0, _CHUNK_ROWS // 16, grp, carry)

    lax.fori_loop(0, _NCHUNK, chunk_body, 0)
    pltpu.sync_copy(out_v, out_hbm.at[pl.ds(wid * _ROWS_W, _ROWS_W)])


def _gather_sum(v, xf):
    f = pl.kernel(
        _gather_body,
        mesh=plsc.VectorSubcoreMesh(core_axis_name="c", subcore_axis_name="s"),
        compiler_params=pltpu.CompilerParams(needs_layout_passes=False),
        out_type=jax.ShapeDtypeStruct((_B,), jnp.float32),
        scratch_types=[
            pltpu.VMEM((_CHUNK,), jnp.int32),
            pltpu.VMEM((_CHUNK,), jnp.float32),
            pltpu.VMEM((_ROWS_W,), jnp.float32),
            pltpu.SemaphoreType.DMA,
        ],
    )
    return f(v, xf)


def kernel(x, table, W, b):
    v = _matvec(table, W, b.reshape(1, 1)).reshape(_VOCAB)
    sums = _gather_sum(v, x.reshape(-1))
    return sums.reshape(_B, 1)
